# baseline (speedup 1.0000x reference)
import functools

import jax
import jax.numpy as jnp
from jax import lax
from jax.experimental import pallas as pl
from jax.experimental.pallas import tpu as pltpu
from jax.experimental.pallas import tpu_sc as plsc

B, N, D = 4, 2048, 2048
K = 10
R = 5
TM = 512
TN = 512


def _stats_body(q_ref, k_ref, p_ref, m_ref, l_ref):
    i = pl.program_id(1)
    qt = q_ref[0]
    rows = lax.broadcasted_iota(jnp.int32, (TM, TN), 0)
    cols = lax.broadcasted_iota(jnp.int32, (TM, TN), 1)
    m = jnp.full((TM,), -3e38, jnp.float32)
    l = jnp.zeros((TM,), jnp.float32)
    diag = jnp.zeros((TM,), jnp.float32)
    for j in range(N // TN):
        kj = k_ref[0, pl.ds(j * TN, TN), :]
        s = lax.dot_general(qt, kj, (((1,), (1,)), ((), ())),
                            preferred_element_type=jnp.float32)
        mj = jnp.max(s, axis=1)
        mn = jnp.maximum(m, mj)
        l = l * jnp.exp(m - mn) + jnp.sum(jnp.exp(s - mn[:, None]), axis=1)
        dmask = (cols + j * TN) == (rows + i * TM)
        diag = diag + jnp.sum(jnp.where(dmask, s, 0.0), axis=1)
        m = mn
    p_ref[0, 0, 0] = jnp.exp(diag - m) / l
    m_ref[0, 0, 0] = m
    l_ref[0, 0, 0] = l


def _diag_stats(q, k):
    shp = jax.ShapeDtypeStruct((B, N // TM, 1, TM), jnp.float32)
    spec = pl.BlockSpec((1, 1, 1, TM), lambda b, i: (b, i, 0, 0))
    p, m, l = pl.pallas_call(
        _stats_body,
        grid=(B, N // TM),
        in_specs=[
            pl.BlockSpec((1, TM, D), lambda b, i: (b, i, 0)),
            pl.BlockSpec((1, N, D), lambda b, i: (b, 0, 0)),
        ],
        out_specs=[spec, spec, spec],
        out_shape=[shp, shp, shp],
    )(q, k)
    return p.reshape(B, N), m.reshape(B, N), l.reshape(B, N)


def _sc_topk_body(p_hbm, m_hbm, l_hbm, ti_hbm, msel_hbm, lsel_hbm,
                  ds_v, tiv_v, mrow_v, lrow_v, msel_v, lsel_v, sem):
    c = lax.axis_index("c")
    s = lax.axis_index("s")
    lane = lax.iota(jnp.int32, 16)

    @pl.when((s < B) & (c == 0))
    def _():
        b = s
        pltpu.sync_copy(p_hbm.at[b], ds_v)

        def merge(t, carry):
            bv, bi = carry
            v = ds_v[pl.ds(t * 16, 16)]
            idx = lane + t * 16
            va, ia = plsc.sort_key_val(v, idx)
            take = va > bv
            mv = jnp.where(take, va, bv)
            mi = jnp.where(take, ia, bi)
            nv, ni = plsc.sort_key_val(mv, mi, descending=True)
            return (nv, ni)

        init = (jnp.full((16,), -1.0, jnp.float32), lane)
        _, bi = lax.fori_loop(0, N // 16, merge, init)
        key = jnp.where(lane < K, bi, jnp.int32(1 << 30))
        si, _ = plsc.sort_key_val(key, key)
        ti = jnp.where(lane < K, si, 0)
        tiv_v[...] = ti
        pltpu.sync_copy(m_hbm.at[b], mrow_v)
        pltpu.sync_copy(l_hbm.at[b], lrow_v)
        msel_v[...] = plsc.load_gather(mrow_v, [ti])
        lsel_v[...] = plsc.load_gather(lrow_v, [ti])
        pltpu.sync_copy(tiv_v, ti_hbm.at[b])
        pltpu.sync_copy(msel_v, msel_hbm.at[b])
        pltpu.sync_copy(lsel_v, lsel_hbm.at[b])


def _sc_assemble_body(ti_hbm, msel_hbm, lsel_hbm, q_hbm, k_hbm,
                      subj_hbm, obj_hbm, rel_hbm,
                      tiv_v, msel_v, lsel_v, ksel_v, obj_v, qa_v, outb_v, sem):
    c = lax.axis_index("c")
    s = lax.axis_index("s")
    wid = s * 2 + c
    lane = lax.iota(jnp.int32, 16)

    for r in range(2):
        item = wid + 32 * r

        @pl.when(item < B * K)
        def _():
            b = item // K
            a = item % K
            pltpu.sync_copy(ti_hbm.at[b], tiv_v)
            pltpu.sync_copy(msel_hbm.at[b], msel_v)
            pltpu.sync_copy(lsel_hbm.at[b], lsel_v)
            ti = tiv_v[...]
            gq = ti + b * N
            ga = jnp.sum(jnp.where(lane == a, ti, 0)) + b * N
            hk = pltpu.async_copy(k_hbm.at[gq], ksel_v, sem)
            pltpu.sync_copy(q_hbm.at[ga], qa_v)
            hk.wait()

            def rnd(x):
                bits = plsc.bitcast(x, jnp.int32)
                bits = (bits + 0x7FFF + ((bits >> 16) & 1)) & jnp.int32(-65536)
                return plsc.bitcast(bits, jnp.float32)

            def dot_body(t, accs):
                qv = rnd(qa_v[pl.ds(t * 16, 16)])
                return tuple(accs[cc] + qv * rnd(ksel_v[cc, pl.ds(t * 16, 16)])
                             for cc in range(K))

            zero = jnp.zeros((16,), jnp.float32)
            accs = lax.fori_loop(0, D // 16, dot_body, (zero,) * K)
            m_a = jnp.sum(jnp.where(lane == a, msel_v[...], 0.0))
            l_a = jnp.sum(jnp.where(lane == a, lsel_v[...], 0.0))
            svec = jnp.full((16,), -3e38, jnp.float32)
            for cc in range(K):
                svec = jnp.where(lane == cc, jnp.sum(accs[cc]), svec)
            relv = jnp.where(lane < K, jnp.exp(svec - m_a) / l_a, -1.0)
            selm = lane < 0
            work = relv
            for _j in range(R):
                mx = jnp.max(work)
                cj = jnp.min(jnp.where(work == mx, lane, jnp.int32(99)))
                selm = selm | (lane == cj)
                work = jnp.where(lane == cj, -1.0, work)
            ckeys = jnp.where(selm, lane, jnp.int32(99))
            csort, _ = plsc.sort_key_val(ckeys, ckeys)
            govec = plsc.load_gather(tiv_v, [jnp.minimum(csort, 15)])
            gobj = jnp.where(lane < R, govec, 0) + b * N
            pltpu.async_copy(q_hbm.at[gobj], obj_v, sem).wait()

            def add_body(t, carry):
                o = t * 16
                qv = qa_v[pl.ds(o, 16)]
                for j in range(R):
                    ov = obj_v[j, pl.ds(o, 16)]
                    outb_v[pl.ds(j * D + o, 16)] = qv + ov
                    outb_v[pl.ds((R + j) * D + o, 16)] = qv
                    outb_v[pl.ds((2 * R + j) * D + o, 16)] = ov
                return carry

            lax.fori_loop(0, D // 16, add_body, 0)
            g0 = (b * K + a) * R * D
            pltpu.sync_copy(outb_v.at[pl.ds(R * D, R * D)],
                            subj_hbm.at[pl.ds(g0, R * D)])
            pltpu.sync_copy(outb_v.at[pl.ds(2 * R * D, R * D)],
                            obj_hbm.at[pl.ds(g0, R * D)])
            pltpu.sync_copy(outb_v.at[pl.ds(0, R * D)],
                            rel_hbm.at[pl.ds(g0, R * D)])


_MESH = plsc.VectorSubcoreMesh(core_axis_name="c", subcore_axis_name="s")
_CP = pltpu.CompilerParams(needs_layout_passes=False)


def _sc_post(p, m, l, q2, k2):
    topk = pl.kernel(
        _sc_topk_body,
        out_type=[jax.ShapeDtypeStruct((B, 16), jnp.int32),
                  jax.ShapeDtypeStruct((B, 16), jnp.float32),
                  jax.ShapeDtypeStruct((B, 16), jnp.float32)],
        mesh=_MESH,
        compiler_params=_CP,
        scratch_types=[
            pltpu.VMEM((N,), jnp.float32),
            pltpu.VMEM((16,), jnp.int32),
            pltpu.VMEM((N,), jnp.float32),
            pltpu.VMEM((N,), jnp.float32),
            pltpu.VMEM((16,), jnp.float32),
            pltpu.VMEM((16,), jnp.float32),
            pltpu.SemaphoreType.DMA,
        ],
    )
    ti, msel, lsel = topk(p, m, l)
    shp = jax.ShapeDtypeStruct((B * K * R * D,), jnp.float32)
    assemble = pl.kernel(
        _sc_assemble_body,
        out_type=[shp, shp, shp],
        mesh=_MESH,
        compiler_params=_CP,
        scratch_types=[
            pltpu.VMEM((16,), jnp.int32),
            pltpu.VMEM((16,), jnp.float32),
            pltpu.VMEM((16,), jnp.float32),
            pltpu.VMEM((16, D), jnp.float32),
            pltpu.VMEM((16, D), jnp.float32),
            pltpu.VMEM((D,), jnp.float32),
            pltpu.VMEM((3 * R * D,), jnp.float32),
            pltpu.SemaphoreType.DMA,
        ],
    )
    subj, obj, rel = assemble(ti, msel, lsel, q2, k2)
    shape3 = (B, K * R, D)
    return (subj.reshape(shape3), obj.reshape(shape3), rel.reshape(shape3))


def kernel(q, k, top_k_instances, top_k_relationships):
    p_diag, m, l = _diag_stats(q, k)
    q2 = q.reshape(B * N, D)
    k2 = k.reshape(B * N, D)
    return _sc_post(p_diag, m, l, q2, k2)


# speedup vs baseline: 1.0219x; 1.0219x over previous
import functools

import jax
import jax.numpy as jnp
from jax import lax
from jax.experimental import pallas as pl
from jax.experimental.pallas import tpu as pltpu
from jax.experimental.pallas import tpu_sc as plsc

B, N, D = 4, 2048, 2048
K = 10
R = 5
TM = 512
TN = 512


def _stats_body(q_ref, k_ref, p_ref, m_ref, l_ref):
    i = pl.program_id(1)
    qt = q_ref[0]
    rows = lax.broadcasted_iota(jnp.int32, (TM, TN), 0)
    cols = lax.broadcasted_iota(jnp.int32, (TM, TN), 1)
    m = jnp.full((TM,), -3e38, jnp.float32)
    l = jnp.zeros((TM,), jnp.float32)
    diag = jnp.zeros((TM,), jnp.float32)
    for j in range(N // TN):
        kj = k_ref[0, pl.ds(j * TN, TN), :]
        s = lax.dot_general(qt, kj, (((1,), (1,)), ((), ())),
                            preferred_element_type=jnp.float32)
        mj = jnp.max(s, axis=1)
        mn = jnp.maximum(m, mj)
        l = l * jnp.exp(m - mn) + jnp.sum(jnp.exp(s - mn[:, None]), axis=1)
        dmask = (cols + j * TN) == (rows + i * TM)
        diag = diag + jnp.sum(jnp.where(dmask, s, 0.0), axis=1)
        m = mn
    p_ref[0, 0, 0] = jnp.exp(diag - m) / l
    m_ref[0, 0, 0] = m
    l_ref[0, 0, 0] = l


def _diag_stats(q, k):
    shp = jax.ShapeDtypeStruct((B, N // TM, 1, TM), jnp.float32)
    spec = pl.BlockSpec((1, 1, 1, TM), lambda b, i: (b, i, 0, 0))
    p, m, l = pl.pallas_call(
        _stats_body,
        grid=(B, N // TM),
        in_specs=[
            pl.BlockSpec((1, TM, D), lambda b, i: (b, i, 0)),
            pl.BlockSpec((1, N, D), lambda b, i: (b, 0, 0)),
        ],
        out_specs=[spec, spec, spec],
        out_shape=[shp, shp, shp],
    )(q, k)
    return p.reshape(B, N), m.reshape(B, N), l.reshape(B, N)


def _sc_topk_body(p_hbm, m_hbm, l_hbm, ti_hbm, msel_hbm, lsel_hbm,
                  ds_v, tiv_v, mrow_v, lrow_v, msel_v, lsel_v, sem):
    c = lax.axis_index("c")
    s = lax.axis_index("s")
    lane = lax.iota(jnp.int32, 16)

    @pl.when((s < B) & (c == 0))
    def _():
        b = s
        pltpu.sync_copy(p_hbm.at[b], ds_v)

        def merge(t, carry):
            bv, bi = carry
            v = ds_v[pl.ds(t * 16, 16)]
            idx = lane + t * 16
            va, ia = plsc.sort_key_val(v, idx)
            take = va > bv
            mv = jnp.where(take, va, bv)
            mi = jnp.where(take, ia, bi)
            nv, ni = plsc.sort_key_val(mv, mi, descending=True)
            return (nv, ni)

        init = (jnp.full((16,), -1.0, jnp.float32), lane)
        _, bi = lax.fori_loop(0, N // 16, merge, init)
        key = jnp.where(lane < K, bi, jnp.int32(1 << 30))
        si, _ = plsc.sort_key_val(key, key)
        ti = jnp.where(lane < K, si, 0)
        tiv_v[...] = ti
        pltpu.sync_copy(m_hbm.at[b], mrow_v)
        pltpu.sync_copy(l_hbm.at[b], lrow_v)
        msel_v[...] = plsc.load_gather(mrow_v, [ti])
        lsel_v[...] = plsc.load_gather(lrow_v, [ti])
        pltpu.sync_copy(tiv_v, ti_hbm.at[b])
        pltpu.sync_copy(msel_v, msel_hbm.at[b])
        pltpu.sync_copy(lsel_v, lsel_hbm.at[b])


def _sc_assemble_body(ti_hbm, msel_hbm, lsel_hbm, q_hbm, k_hbm,
                      subj_hbm, obj_hbm, rel_hbm,
                      tiv_v, msel_v, lsel_v, ksel_v, qa_v, qo_v, rel_v, sem):
    c = lax.axis_index("c")
    s = lax.axis_index("s")
    wid = s * 2 + c
    lane = lax.iota(jnp.int32, 16)

    for r in range(2):
        item = wid + 32 * r

        @pl.when(item < B * K)
        def _():
            b = item // K
            a = item % K
            pltpu.sync_copy(ti_hbm.at[b], tiv_v)
            pltpu.sync_copy(msel_hbm.at[b], msel_v)
            pltpu.sync_copy(lsel_hbm.at[b], lsel_v)
            ti = tiv_v[...]
            gq = ti + b * N
            ga = jnp.sum(jnp.where(lane == a, ti, 0)) + b * N
            hk = pltpu.async_copy(k_hbm.at[gq], ksel_v, sem)
            pltpu.sync_copy(q_hbm.at[ga], qa_v)
            hk.wait()

            def rnd(x):
                bits = plsc.bitcast(x, jnp.int32)
                bits = (bits + 0x7FFF + ((bits >> 16) & 1)) & jnp.int32(-65536)
                return plsc.bitcast(bits, jnp.float32)

            def dot_body(t, accs):
                qv = rnd(qa_v[pl.ds(t * 16, 16)])
                return tuple(accs[cc] + qv * rnd(ksel_v[cc, pl.ds(t * 16, 16)])
                             for cc in range(K))

            zero = jnp.zeros((16,), jnp.float32)
            accs = lax.fori_loop(0, D // 16, dot_body, (zero,) * K)
            m_a = jnp.sum(jnp.where(lane == a, msel_v[...], 0.0))
            l_a = jnp.sum(jnp.where(lane == a, lsel_v[...], 0.0))
            svec = jnp.full((16,), -3e38, jnp.float32)
            for cc in range(K):
                svec = jnp.where(lane == cc, jnp.sum(accs[cc]), svec)
            relv = jnp.where(lane < K, jnp.exp(svec - m_a) / l_a, -1.0)
            selm = lane < 0
            work = relv
            for _j in range(R):
                mx = jnp.max(work)
                cj = jnp.min(jnp.where(work == mx, lane, jnp.int32(99)))
                selm = selm | (lane == cj)
                work = jnp.where(lane == cj, -1.0, work)
            ckeys = jnp.where(selm, lane, jnp.int32(99))
            csort, _ = plsc.sort_key_val(ckeys, ckeys)
            govec = plsc.load_gather(tiv_v, [jnp.minimum(csort, 15)])
            for j in range(R):
                go = jnp.sum(jnp.where(lane == j, govec, 0)) + b * N
                pltpu.sync_copy(q_hbm.at[go], qo_v)

                def add_body(t, carry):
                    sl = pl.ds(t * 16, 16)
                    rel_v[sl] = qa_v[sl] + qo_v[sl]
                    return carry

                lax.fori_loop(0, D // 16, add_body, 0)
                row = a * R + j
                pltpu.sync_copy(qa_v, subj_hbm.at[b, row])
                pltpu.sync_copy(qo_v, obj_hbm.at[b, row])
                pltpu.sync_copy(rel_v, rel_hbm.at[b, row])


_MESH = plsc.VectorSubcoreMesh(core_axis_name="c", subcore_axis_name="s")
_CP = pltpu.CompilerParams(needs_layout_passes=False)


def _sc_post(p, m, l, q2, k2):
    topk = pl.kernel(
        _sc_topk_body,
        out_type=[jax.ShapeDtypeStruct((B, 16), jnp.int32),
                  jax.ShapeDtypeStruct((B, 16), jnp.float32),
                  jax.ShapeDtypeStruct((B, 16), jnp.float32)],
        mesh=_MESH,
        compiler_params=_CP,
        scratch_types=[
            pltpu.VMEM((N,), jnp.float32),
            pltpu.VMEM((16,), jnp.int32),
            pltpu.VMEM((N,), jnp.float32),
            pltpu.VMEM((N,), jnp.float32),
            pltpu.VMEM((16,), jnp.float32),
            pltpu.VMEM((16,), jnp.float32),
            pltpu.SemaphoreType.DMA,
        ],
    )
    ti, msel, lsel = topk(p, m, l)
    shp = jax.ShapeDtypeStruct((B, K * R, D), jnp.float32)
    assemble = pl.kernel(
        _sc_assemble_body,
        out_type=[shp, shp, shp],
        mesh=_MESH,
        compiler_params=_CP,
        scratch_types=[
            pltpu.VMEM((16,), jnp.int32),
            pltpu.VMEM((16,), jnp.float32),
            pltpu.VMEM((16,), jnp.float32),
            pltpu.VMEM((16, D), jnp.float32),
            pltpu.VMEM((D,), jnp.float32),
            pltpu.VMEM((D,), jnp.float32),
            pltpu.VMEM((D,), jnp.float32),
            pltpu.SemaphoreType.DMA,
        ],
    )
    return tuple(assemble(ti, msel, lsel, q2, k2))


def kernel(q, k, top_k_instances, top_k_relationships):
    p_diag, m, l = _diag_stats(q, k)
    q2 = q.reshape(B * N, D)
    k2 = k.reshape(B * N, D)
    return _sc_post(p_diag, m, l, q2, k2)


# speedup vs baseline: 1.0728x; 1.0498x over previous
import functools

import jax
import jax.numpy as jnp
from jax import lax
from jax.experimental import pallas as pl
from jax.experimental.pallas import tpu as pltpu
from jax.experimental.pallas import tpu_sc as plsc

B, N, D = 4, 2048, 2048
K = 10
R = 5
TM = 512
TN = 512


def _stats_body(q_ref, k_ref, p_ref, m_ref, l_ref):
    i = pl.program_id(1)
    qt = q_ref[0]
    rows = lax.broadcasted_iota(jnp.int32, (TM, TN), 0)
    cols = lax.broadcasted_iota(jnp.int32, (TM, TN), 1)
    m = jnp.full((TM,), -3e38, jnp.float32)
    l = jnp.zeros((TM,), jnp.float32)
    diag = jnp.zeros((TM,), jnp.float32)
    for j in range(N // TN):
        kj = k_ref[0, pl.ds(j * TN, TN), :]
        s = lax.dot_general(qt, kj, (((1,), (1,)), ((), ())),
                            preferred_element_type=jnp.float32)
        mj = jnp.max(s, axis=1)
        mn = jnp.maximum(m, mj)
        l = l * jnp.exp(m - mn) + jnp.sum(jnp.exp(s - mn[:, None]), axis=1)
        dmask = (cols + j * TN) == (rows + i * TM)
        diag = diag + jnp.sum(jnp.where(dmask, s, 0.0), axis=1)
        m = mn
    p_ref[0, 0, 0] = jnp.exp(diag - m) / l
    m_ref[0, 0, 0] = m
    l_ref[0, 0, 0] = l


def _diag_stats(q, k):
    shp = jax.ShapeDtypeStruct((B, N // TM, 1, TM), jnp.float32)
    spec = pl.BlockSpec((1, 1, 1, TM), lambda b, i: (b, i, 0, 0))
    p, m, l = pl.pallas_call(
        _stats_body,
        grid=(B, N // TM),
        in_specs=[
            pl.BlockSpec((1, TM, D), lambda b, i: (b, i, 0)),
            pl.BlockSpec((1, N, D), lambda b, i: (b, 0, 0)),
        ],
        out_specs=[spec, spec, spec],
        out_shape=[shp, shp, shp],
    )(q, k)
    return p.reshape(B, N), m.reshape(B, N), l.reshape(B, N)


def _sc_topk_body(p_hbm, m_hbm, l_hbm, ti_hbm, msel_hbm, lsel_hbm,
                  ds_v, tiv_v, mrow_v, lrow_v, msel_v, lsel_v, sem):
    c = lax.axis_index("c")
    s = lax.axis_index("s")
    lane = lax.iota(jnp.int32, 16)

    @pl.when((s < B) & (c == 0))
    def _():
        b = s
        pltpu.sync_copy(p_hbm.at[b], ds_v)

        def merge(t, carry):
            bv, bi = carry
            v = ds_v[pl.ds(t * 16, 16)]
            idx = lane + t * 16
            va, ia = plsc.sort_key_val(v, idx)
            take = va > bv
            mv = jnp.where(take, va, bv)
            mi = jnp.where(take, ia, bi)
            nv, ni = plsc.sort_key_val(mv, mi, descending=True)
            return (nv, ni)

        init = (jnp.full((16,), -1.0, jnp.float32), lane)
        _, bi = lax.fori_loop(0, N // 16, merge, init)
        key = jnp.where(lane < K, bi, jnp.int32(1 << 30))
        si, _ = plsc.sort_key_val(key, key)
        ti = jnp.where(lane < K, si, 0)
        tiv_v[...] = ti
        pltpu.sync_copy(m_hbm.at[b], mrow_v)
        pltpu.sync_copy(l_hbm.at[b], lrow_v)
        msel_v[...] = plsc.load_gather(mrow_v, [ti])
        lsel_v[...] = plsc.load_gather(lrow_v, [ti])
        pltpu.sync_copy(tiv_v, ti_hbm.at[b])
        pltpu.sync_copy(msel_v, msel_hbm.at[b])
        pltpu.sync_copy(lsel_v, lsel_hbm.at[b])


def _sc_assemble_body(ti_hbm, msel_hbm, lsel_hbm, q_hbm, k_hbm,
                      subj_hbm, obj_hbm, rel_hbm,
                      tiv_v, msel_v, lsel_v, ksel_v, qa_v, qo_v, qo2_v, rel_v, rel2_v, sem):
    c = lax.axis_index("c")
    s = lax.axis_index("s")
    wid = s * 2 + c
    lane = lax.iota(jnp.int32, 16)

    for r in range(2):
        item = wid + 32 * r

        @pl.when(item < B * K)
        def _():
            b = item // K
            a = item % K
            pltpu.sync_copy(ti_hbm.at[b], tiv_v)
            pltpu.sync_copy(msel_hbm.at[b], msel_v)
            pltpu.sync_copy(lsel_hbm.at[b], lsel_v)
            ti = tiv_v[...]
            gq = ti + b * N
            ga = jnp.sum(jnp.where(lane == a, ti, 0)) + b * N
            hk = pltpu.async_copy(k_hbm.at[gq], ksel_v, sem)
            pltpu.sync_copy(q_hbm.at[ga], qa_v)
            hk.wait()

            def rnd(x):
                bits = plsc.bitcast(x, jnp.int32)
                bits = (bits + 0x7FFF + ((bits >> 16) & 1)) & jnp.int32(-65536)
                return plsc.bitcast(bits, jnp.float32)

            def dot_body(t, accs):
                qv = rnd(qa_v[pl.ds(t * 16, 16)])
                return tuple(accs[cc] + qv * rnd(ksel_v[cc, pl.ds(t * 16, 16)])
                             for cc in range(K))

            zero = jnp.zeros((16,), jnp.float32)
            accs = lax.fori_loop(0, D // 16, dot_body, (zero,) * K)
            m_a = jnp.sum(jnp.where(lane == a, msel_v[...], 0.0))
            l_a = jnp.sum(jnp.where(lane == a, lsel_v[...], 0.0))
            svec = jnp.full((16,), -3e38, jnp.float32)
            for cc in range(K):
                svec = jnp.where(lane == cc, jnp.sum(accs[cc]), svec)
            relv = jnp.where(lane < K, jnp.exp(svec - m_a) / l_a, -1.0)
            selm = lane < 0
            work = relv
            for _j in range(R):
                mx = jnp.max(work)
                cj = jnp.min(jnp.where(work == mx, lane, jnp.int32(99)))
                selm = selm | (lane == cj)
                work = jnp.where(lane == cj, -1.0, work)
            ckeys = jnp.where(selm, lane, jnp.int32(99))
            csort, _ = plsc.sort_key_val(ckeys, ckeys)
            govec = plsc.load_gather(tiv_v, [jnp.minimum(csort, 15)])

            def go_row(j):
                return jnp.sum(jnp.where(lane == j, govec, 0)) + b * N

            qbufs = [qo_v, qo2_v]
            rbufs = [rel_v, rel2_v]
            hq = pltpu.async_copy(q_hbm.at[go_row(0)], qbufs[0], sem)
            outs = []
            for j in range(R):
                hq.wait()
                cur = qbufs[j % 2]
                crl = rbufs[j % 2]
                if j + 1 < R:
                    hq = pltpu.async_copy(q_hbm.at[go_row(j + 1)],
                                          qbufs[(j + 1) % 2], sem)

                def add_body(t, carry, cur=cur, crl=crl):
                    sl = pl.ds(t * 16, 16)
                    crl[sl] = qa_v[sl] + cur[sl]
                    return carry

                lax.fori_loop(0, D // 16, add_body, 0)
                row = a * R + j
                outs.append(pltpu.async_copy(qa_v, subj_hbm.at[b, row], sem))
                outs.append(pltpu.async_copy(cur, obj_hbm.at[b, row], sem))
                outs.append(pltpu.async_copy(crl, rel_hbm.at[b, row], sem))
            for h in outs:
                h.wait()


_MESH = plsc.VectorSubcoreMesh(core_axis_name="c", subcore_axis_name="s")
_CP = pltpu.CompilerParams(needs_layout_passes=False)


def _sc_post(p, m, l, q2, k2):
    topk = pl.kernel(
        _sc_topk_body,
        out_type=[jax.ShapeDtypeStruct((B, 16), jnp.int32),
                  jax.ShapeDtypeStruct((B, 16), jnp.float32),
                  jax.ShapeDtypeStruct((B, 16), jnp.float32)],
        mesh=_MESH,
        compiler_params=_CP,
        scratch_types=[
            pltpu.VMEM((N,), jnp.float32),
            pltpu.VMEM((16,), jnp.int32),
            pltpu.VMEM((N,), jnp.float32),
            pltpu.VMEM((N,), jnp.float32),
            pltpu.VMEM((16,), jnp.float32),
            pltpu.VMEM((16,), jnp.float32),
            pltpu.SemaphoreType.DMA,
        ],
    )
    ti, msel, lsel = topk(p, m, l)
    shp = jax.ShapeDtypeStruct((B, K * R, D), jnp.float32)
    assemble = pl.kernel(
        _sc_assemble_body,
        out_type=[shp, shp, shp],
        mesh=_MESH,
        compiler_params=_CP,
        scratch_types=[
            pltpu.VMEM((16,), jnp.int32),
            pltpu.VMEM((16,), jnp.float32),
            pltpu.VMEM((16,), jnp.float32),
            pltpu.VMEM((16, D), jnp.float32),
            pltpu.VMEM((D,), jnp.float32),
            pltpu.VMEM((D,), jnp.float32),
            pltpu.VMEM((D,), jnp.float32),
            pltpu.VMEM((D,), jnp.float32),
            pltpu.VMEM((D,), jnp.float32),
            pltpu.SemaphoreType.DMA,
        ],
    )
    return tuple(assemble(ti, msel, lsel, q2, k2))


def kernel(q, k, top_k_instances, top_k_relationships):
    p_diag, m, l = _diag_stats(q, k)
    q2 = q.reshape(B * N, D)
    k2 = k.reshape(B * N, D)
    return _sc_post(p_diag, m, l, q2, k2)


# speedup vs baseline: 1.0733x; 1.0005x over previous
import functools

import jax
import jax.numpy as jnp
from jax import lax
from jax.experimental import pallas as pl
from jax.experimental.pallas import tpu as pltpu
from jax.experimental.pallas import tpu_sc as plsc

B, N, D = 4, 2048, 2048
K = 10
R = 5
TM = 512
TN = 512


def _stats_body(q_ref, k_ref, p_ref, m_ref, l_ref):
    i = pl.program_id(1)
    qt = q_ref[0]
    rows = lax.broadcasted_iota(jnp.int32, (TM, TN), 0)
    cols = lax.broadcasted_iota(jnp.int32, (TM, TN), 1)
    m = jnp.full((TM,), -3e38, jnp.float32)
    l = jnp.zeros((TM,), jnp.float32)
    diag = jnp.zeros((TM,), jnp.float32)
    for j in range(N // TN):
        kj = k_ref[0, pl.ds(j * TN, TN), :]
        s = lax.dot_general(qt, kj, (((1,), (1,)), ((), ())),
                            preferred_element_type=jnp.float32)
        mj = jnp.max(s, axis=1)
        mn = jnp.maximum(m, mj)
        l = l * jnp.exp(m - mn) + jnp.sum(jnp.exp(s - mn[:, None]), axis=1)
        dmask = (cols + j * TN) == (rows + i * TM)
        diag = diag + jnp.sum(jnp.where(dmask, s, 0.0), axis=1)
        m = mn
    p_ref[0, 0, 0] = jnp.exp(diag - m) / l
    m_ref[0, 0, 0] = m
    l_ref[0, 0, 0] = l


def _diag_stats(q, k):
    shp = jax.ShapeDtypeStruct((B, N // TM, 1, TM), jnp.float32)
    spec = pl.BlockSpec((1, 1, 1, TM), lambda b, i: (b, i, 0, 0))
    p, m, l = pl.pallas_call(
        _stats_body,
        grid=(B, N // TM),
        in_specs=[
            pl.BlockSpec((1, TM, D), lambda b, i: (b, i, 0)),
            pl.BlockSpec((1, N, D), lambda b, i: (b, 0, 0)),
        ],
        out_specs=[spec, spec, spec],
        out_shape=[shp, shp, shp],
    )(q, k)
    return p.reshape(B, N), m.reshape(B, N), l.reshape(B, N)


def _sc_topk_body(p_hbm, m_hbm, l_hbm, ti_hbm, msel_hbm, lsel_hbm,
                  ds_v, tiv_v, mrow_v, lrow_v, msel_v, lsel_v, sem):
    c = lax.axis_index("c")
    s = lax.axis_index("s")
    lane = lax.iota(jnp.int32, 16)

    @pl.when((s < B) & (c == 0))
    def _():
        b = s
        pltpu.sync_copy(p_hbm.at[b], ds_v)

        def merge(t, carry):
            bv, bi = carry
            v = ds_v[pl.ds(t * 16, 16)]
            idx = lane + t * 16
            va, ia = plsc.sort_key_val(v, idx)
            take = va > bv
            mv = jnp.where(take, va, bv)
            mi = jnp.where(take, ia, bi)
            nv, ni = plsc.sort_key_val(mv, mi, descending=True)
            return (nv, ni)

        init = (jnp.full((16,), -1.0, jnp.float32), lane)
        _, bi = lax.fori_loop(0, N // 16, merge, init)
        key = jnp.where(lane < K, bi, jnp.int32(1 << 30))
        si, _ = plsc.sort_key_val(key, key)
        ti = jnp.where(lane < K, si, 0)
        tiv_v[...] = ti
        pltpu.sync_copy(m_hbm.at[b], mrow_v)
        pltpu.sync_copy(l_hbm.at[b], lrow_v)
        msel_v[...] = plsc.load_gather(mrow_v, [ti])
        lsel_v[...] = plsc.load_gather(lrow_v, [ti])
        pltpu.sync_copy(tiv_v, ti_hbm.at[b])
        pltpu.sync_copy(msel_v, msel_hbm.at[b])
        pltpu.sync_copy(lsel_v, lsel_hbm.at[b])


def _sc_assemble_body(ti_hbm, msel_hbm, lsel_hbm, q_hbm, k_hbm,
                      subj_hbm, obj_hbm, rel_hbm,
                      tiv_v, msel_v, lsel_v, ksel_v, qa_v, qo_bufs, rl_bufs, sem):
    c = lax.axis_index("c")
    s = lax.axis_index("s")
    wid = s * 2 + c
    lane = lax.iota(jnp.int32, 16)

    for r in range(2):
        item = wid + 32 * r

        @pl.when(item < B * K)
        def _():
            b = item // K
            a = item % K
            pltpu.sync_copy(ti_hbm.at[b], tiv_v)
            pltpu.sync_copy(msel_hbm.at[b], msel_v)
            pltpu.sync_copy(lsel_hbm.at[b], lsel_v)
            ti = tiv_v[...]
            gq = ti + b * N
            ga = jnp.sum(jnp.where(lane == a, ti, 0)) + b * N
            hk = pltpu.async_copy(k_hbm.at[gq], ksel_v, sem)
            pltpu.sync_copy(q_hbm.at[ga], qa_v)
            hk.wait()

            def rnd(x):
                bits = plsc.bitcast(x, jnp.int32)
                bits = (bits + 0x7FFF + ((bits >> 16) & 1)) & jnp.int32(-65536)
                return plsc.bitcast(bits, jnp.float32)

            def dot_body(t, accs):
                qv = rnd(qa_v[pl.ds(t * 16, 16)])
                return tuple(accs[cc] + qv * rnd(ksel_v[cc, pl.ds(t * 16, 16)])
                             for cc in range(K))

            zero = jnp.zeros((16,), jnp.float32)
            accs = lax.fori_loop(0, D // 16, dot_body, (zero,) * K)
            m_a = jnp.sum(jnp.where(lane == a, msel_v[...], 0.0))
            l_a = jnp.sum(jnp.where(lane == a, lsel_v[...], 0.0))
            svec = jnp.full((16,), -3e38, jnp.float32)
            for cc in range(K):
                svec = jnp.where(lane == cc, jnp.sum(accs[cc]), svec)
            relv = jnp.where(lane < K, jnp.exp(svec - m_a) / l_a, -1.0)
            selm = lane < 0
            work = relv
            for _j in range(R):
                mx = jnp.max(work)
                cj = jnp.min(jnp.where(work == mx, lane, jnp.int32(99)))
                selm = selm | (lane == cj)
                work = jnp.where(lane == cj, -1.0, work)
            ckeys = jnp.where(selm, lane, jnp.int32(99))
            csort, _ = plsc.sort_key_val(ckeys, ckeys)
            govec = plsc.load_gather(tiv_v, [jnp.minimum(csort, 15)])

            def go_row(j):
                return jnp.sum(jnp.where(lane == j, govec, 0)) + b * N

            hq = pltpu.async_copy(q_hbm.at[go_row(0)], qo_bufs[0], sem)
            outs = []
            for j in range(R):
                hq.wait()
                cur = qo_bufs[j]
                crl = rl_bufs[j]
                if j + 1 < R:
                    hq = pltpu.async_copy(q_hbm.at[go_row(j + 1)],
                                          qo_bufs[j + 1], sem)

                def add_body(t, carry, cur=cur, crl=crl):
                    sl = pl.ds(t * 16, 16)
                    crl[sl] = qa_v[sl] + cur[sl]
                    return carry

                lax.fori_loop(0, D // 16, add_body, 0)
                row = a * R + j
                outs.append(pltpu.async_copy(qa_v, subj_hbm.at[b, row], sem))
                outs.append(pltpu.async_copy(cur, obj_hbm.at[b, row], sem))
                outs.append(pltpu.async_copy(crl, rel_hbm.at[b, row], sem))
            for h in outs:
                h.wait()


_MESH = plsc.VectorSubcoreMesh(core_axis_name="c", subcore_axis_name="s")
_CP = pltpu.CompilerParams(needs_layout_passes=False)


def _sc_post(p, m, l, q2, k2):
    topk = pl.kernel(
        _sc_topk_body,
        out_type=[jax.ShapeDtypeStruct((B, 16), jnp.int32),
                  jax.ShapeDtypeStruct((B, 16), jnp.float32),
                  jax.ShapeDtypeStruct((B, 16), jnp.float32)],
        mesh=_MESH,
        compiler_params=_CP,
        scratch_types=[
            pltpu.VMEM((N,), jnp.float32),
            pltpu.VMEM((16,), jnp.int32),
            pltpu.VMEM((N,), jnp.float32),
            pltpu.VMEM((N,), jnp.float32),
            pltpu.VMEM((16,), jnp.float32),
            pltpu.VMEM((16,), jnp.float32),
            pltpu.SemaphoreType.DMA,
        ],
    )
    ti, msel, lsel = topk(p, m, l)
    shp = jax.ShapeDtypeStruct((B, K * R, D), jnp.float32)
    assemble = pl.kernel(
        _sc_assemble_body,
        out_type=[shp, shp, shp],
        mesh=_MESH,
        compiler_params=_CP,
        scratch_types=[
            pltpu.VMEM((16,), jnp.int32),
            pltpu.VMEM((16,), jnp.float32),
            pltpu.VMEM((16,), jnp.float32),
            pltpu.VMEM((16, D), jnp.float32),
            pltpu.VMEM((D,), jnp.float32),
            tuple(pltpu.VMEM((D,), jnp.float32) for _ in range(R)),
            tuple(pltpu.VMEM((D,), jnp.float32) for _ in range(R)),
            pltpu.SemaphoreType.DMA,
        ],
    )
    return tuple(assemble(ti, msel, lsel, q2, k2))


def kernel(q, k, top_k_instances, top_k_relationships):
    p_diag, m, l = _diag_stats(q, k)
    q2 = q.reshape(B * N, D)
    k2 = k.reshape(B * N, D)
    return _sc_post(p_diag, m, l, q2, k2)
